# Initial kernel scaffold; baseline (speedup 1.0000x reference)
import jax
import jax.numpy as jnp
from jax.experimental import pallas as pl


def kernel(x, edge_index, prop_nums, W1, b1, Wc, bc):
    raise NotImplementedError("write your pallas kernel here")


# speedup vs baseline: 3.2065x; 3.2065x over previous
import functools

import jax
import jax.numpy as jnp
from jax import lax
from jax.experimental import pallas as pl
from jax.experimental.pallas import tpu as pltpu
from jax.experimental.pallas import tpu_sc as plsc

N = 10000
NPAD = 10240
HALF = 5000
TRASH = HALF
ACC_ROWS = 5120
FIN = 320
FINB = 128

B = 128
SB = 8
NSB = 20
EPT = B * SB * NSB
E_PAD = 16 * EPT
EROWS = E_PAD // B

_mesh = lambda: plsc.VectorSubcoreMesh(core_axis_name="c", subcore_axis_name="s")


def _dinv_tc_call(deg):
    BM = 2000

    def body(deg_ref, dinv_ref, d2_ref):
        y = lax.rsqrt(deg_ref[:, :16])
        dinv_ref[...] = y
        d2_ref[...] = y * y

    return pl.pallas_call(
        body,
        out_shape=(
            jax.ShapeDtypeStruct((N, 16), jnp.float32),
            jax.ShapeDtypeStruct((N, 16), jnp.float32),
        ),
        grid=(N // BM,),
        in_specs=[pl.BlockSpec((BM, 128), lambda i: (i, 0))],
        out_specs=(
            pl.BlockSpec((BM, 16), lambda i: (i, 0)),
            pl.BlockSpec((BM, 16), lambda i: (i, 0)),
        ),
    )(deg)


def _make_prop(D):
    @functools.partial(
        pl.kernel,
        out_type=jax.ShapeDtypeStruct((N, D), jnp.float32),
        mesh=_mesh(),
        scratch_types=[
            pltpu.VMEM_SHARED((ACC_ROWS, D), jnp.float32),
            pltpu.VMEM((SB, B), jnp.int32),
            pltpu.VMEM((SB, B), jnp.int32),
            pltpu.VMEM((SB, B), jnp.int32),
            pltpu.VMEM((B, D), jnp.float32),
            pltpu.VMEM((B, D), jnp.float32),
            pltpu.VMEM((FINB, D), jnp.float32),
            pltpu.VMEM((FINB, D), jnp.float32),
            pltpu.VMEM((FINB, 16), jnp.float32),
            pltpu.SemaphoreType.DMA,
            pltpu.SemaphoreType.DMA,
        ],
    )
    def k(g_hbm, src_hbm, dst_hbm, scale_hbm, out_hbm,
          acc, ssrc, sdst, dloc, rows0, rows1, facc, fg, scbuf, sem0, sem1):
        c = lax.axis_index("c")
        s = lax.axis_index("s")
        nodebase = c * HALF
        rowsbufs = (rows0, rows1)
        sems = (sem0, sem1)

        def zrow(i, _):
            for kk in range(D // 16):
                facc[i, pl.ds(kk * 16, 16)] = jnp.zeros((16,), jnp.float32)
            return 0

        lax.fori_loop(0, FINB, zrow, 0)
        pltpu.sync_copy(facc, acc.at[pl.ds(s * FIN, FINB)])
        pltpu.sync_copy(facc, acc.at[pl.ds(s * FIN + FINB, FINB)])
        pltpu.sync_copy(facc.at[pl.ds(0, 64)], acc.at[pl.ds(s * FIN + 2 * FINB, 64)])
        plsc.subcore_barrier()

        def superbatch(j, _):
            erow = s * (NSB * SB) + j * SB
            pltpu.sync_copy(src_hbm.at[pl.ds(erow, SB)], ssrc)
            pltpu.sync_copy(dst_hbm.at[pl.ds(erow, SB)], sdst)
            for b in range(SB):
                for kk in range(B // 16):
                    dvec = sdst[b, pl.ds(kk * 16, 16)]
                    inr = (dvec >= nodebase) & (dvec < nodebase + HALF)
                    dloc[b, pl.ds(kk * 16, 16)] = jnp.where(
                        inr, dvec - nodebase, TRASH)
            descs = [None, None]
            descs[0] = pltpu.async_copy(
                g_hbm.at[ssrc.at[0]], rowsbufs[0], sems[0])
            for b in range(SB):
                p = b % 2
                if b + 1 < SB:
                    descs[1 - p] = pltpu.async_copy(
                        g_hbm.at[ssrc.at[b + 1]], rowsbufs[1 - p], sems[1 - p])
                descs[p].wait()
                pltpu.sync_copy(rowsbufs[p], acc.at[dloc.at[b]], add=True)
            return 0

        lax.fori_loop(0, NSB, superbatch, 0)
        plsc.subcore_barrier()

        def fin(off, sz):
            gbase = nodebase + s * FIN + off
            pltpu.sync_copy(acc.at[pl.ds(s * FIN + off, sz)], facc.at[pl.ds(0, sz)])
            pltpu.sync_copy(g_hbm.at[pl.ds(gbase, sz)], fg.at[pl.ds(0, sz)])
            pltpu.sync_copy(scale_hbm.at[pl.ds(gbase, sz)], scbuf.at[pl.ds(0, sz)])

            def frow(i, _):
                sc = scbuf[i, :][0]
                for kk in range(D // 16):
                    sl = pl.ds(kk * 16, 16)
                    facc[i, sl] = sc * (facc[i, sl] + fg[i, sl])
                return 0

            lax.fori_loop(0, sz, frow, 0)
            pltpu.sync_copy(facc.at[pl.ds(0, sz)], out_hbm.at[pl.ds(gbase, sz)])

        @pl.when(s == 15)
        def _():
            fin(0, FINB)
            fin(FINB, 72)

        @pl.when(s < 15)
        def _():
            fin(0, FINB)
            fin(FINB, FINB)
            fin(2 * FINB, 64)

    return k


def _tc_in_call(x, W1, b1, dinv_col):
    D = x.shape[1]
    H = W1.shape[1]
    BM = 400

    def body(x_ref, w_ref, b_ref, dv_ref, o_ref):
        h = jnp.dot(x_ref[...], w_ref[...], preferred_element_type=jnp.float32)
        o_ref[...] = dv_ref[...] * (h + b_ref[...])

    return pl.pallas_call(
        body,
        out_shape=jax.ShapeDtypeStruct((N, H), jnp.float32),
        grid=(N // BM,),
        in_specs=[
            pl.BlockSpec((BM, D), lambda i: (i, 0)),
            pl.BlockSpec((D, H), lambda i: (0, 0)),
            pl.BlockSpec((1, H), lambda i: (0, 0)),
            pl.BlockSpec((BM, 1), lambda i: (i, 0)),
        ],
        out_specs=pl.BlockSpec((BM, H), lambda i: (i, 0)),
    )(x, W1, b1, dinv_col)


def _tc_out_call(g30, Wc, bc, dinv_col):
    H = g30.shape[1]
    C = Wc.shape[1]
    BM = 400

    def body(g_ref, w_ref, b_ref, dv_ref, o_ref):
        h = jax.nn.relu(g_ref[...] / dv_ref[...])
        o = jnp.dot(h, w_ref[...], preferred_element_type=jnp.float32)
        o_ref[...] = dv_ref[...] * (o + b_ref[...])

    return pl.pallas_call(
        body,
        out_shape=jax.ShapeDtypeStruct((N, C), jnp.float32),
        grid=(N // BM,),
        in_specs=[
            pl.BlockSpec((BM, H), lambda i: (i, 0)),
            pl.BlockSpec((H, C), lambda i: (0, 0)),
            pl.BlockSpec((1, C), lambda i: (0, 0)),
            pl.BlockSpec((BM, 1), lambda i: (i, 0)),
        ],
        out_specs=pl.BlockSpec((BM, C), lambda i: (i, 0)),
    )(g30, Wc, bc, dinv_col)


def kernel(x, edge_index, prop_nums, W1, b1, Wc, bc):
    D = x.shape[1]
    C = Wc.shape[1]
    E = edge_index.shape[1]
    src = edge_index[0].astype(jnp.int32)
    dst = edge_index[1].astype(jnp.int32)
    pad = E_PAD - E
    srcp = jnp.concatenate([src, jnp.zeros((pad,), jnp.int32)])
    dstp = jnp.concatenate([dst, jnp.full((pad,), N, jnp.int32)])
    src2d = srcp.reshape(EROWS, B)
    dst2d = dstp.reshape(EROWS, B)

    prop128 = _make_prop(D)
    b1r = b1.reshape(1, -1)
    bcr = bc.reshape(1, -1)

    def phase_deg(g, dv, d2):
        return (jnp.ones((N, D), jnp.float32),
                jnp.ones((N, 16), jnp.float32), dv, d2)

    def phase_init(g, dv, d2):
        dv2, d22 = _dinv_tc_call(g)
        gin = _tc_in_call(x, W1, b1r, dv2[:, :1])
        return gin, d22, dv2, d22

    def phase_mid(g, dv, d2):
        return g, d2, dv, d2

    def phase_last(g, dv, d2):
        q = _tc_out_call(g, Wc, bcr, dv[:, :1])
        return jnp.pad(q, ((0, 0), (0, D - C))), dv, dv, d2

    def loop_body(i, carry):
        g, dv, d2 = carry
        sel = jnp.where(i == 0, 0,
                        jnp.where(i == 1, 1,
                                  jnp.where(i == prop_nums + 1, 3, 2)))
        gin, scale, dv, d2 = lax.switch(
            sel, [phase_deg, phase_init, phase_mid, phase_last], g, dv, d2)
        return prop128(gin, src2d, dst2d, scale), dv, d2

    zeros16 = jnp.zeros((N, 16), jnp.float32)
    out_full, _, _ = lax.fori_loop(
        0, prop_nums + 2, loop_body, (x, zeros16, zeros16))
    return out_full[:, :C]


# speedup vs baseline: 3.2521x; 1.0142x over previous
import functools

import jax
import jax.numpy as jnp
from jax import lax
from jax.experimental import pallas as pl
from jax.experimental.pallas import tpu as pltpu
from jax.experimental.pallas import tpu_sc as plsc

N = 10000
NPAD = 10240
HALF = 5000
TRASH = HALF
ACC_ROWS = 5120
FIN = 320
FINB = 64

B = 128
SB = 16
NSB = 10
EPT = B * SB * NSB
E_PAD = 16 * EPT
EROWS = E_PAD // B

_mesh = lambda: plsc.VectorSubcoreMesh(core_axis_name="c", subcore_axis_name="s")


def _dinv_tc_call(deg):
    BM = 2000

    def body(deg_ref, dinv_ref, d2_ref):
        y = lax.rsqrt(deg_ref[:, :16])
        dinv_ref[...] = y
        d2_ref[...] = y * y

    return pl.pallas_call(
        body,
        out_shape=(
            jax.ShapeDtypeStruct((N, 16), jnp.float32),
            jax.ShapeDtypeStruct((N, 16), jnp.float32),
        ),
        grid=(N // BM,),
        in_specs=[pl.BlockSpec((BM, 128), lambda i: (i, 0))],
        out_specs=(
            pl.BlockSpec((BM, 16), lambda i: (i, 0)),
            pl.BlockSpec((BM, 16), lambda i: (i, 0)),
        ),
    )(deg)


def _make_prop(D):
    @functools.partial(
        pl.kernel,
        out_type=jax.ShapeDtypeStruct((N, D), jnp.float32),
        mesh=_mesh(),
        scratch_types=[
            pltpu.VMEM_SHARED((ACC_ROWS, D), jnp.float32),
            pltpu.VMEM((SB, B), jnp.int32),
            pltpu.VMEM((SB, B), jnp.int32),
            pltpu.VMEM((SB, B), jnp.int32),
            pltpu.VMEM((SB, B), jnp.int32),
            pltpu.VMEM((SB, B), jnp.int32),
            pltpu.VMEM((SB, B), jnp.int32),
            pltpu.VMEM((B, D), jnp.float32),
            pltpu.VMEM((B, D), jnp.float32),
            pltpu.VMEM((B, D), jnp.float32),
            pltpu.VMEM((FINB, D), jnp.float32),
            pltpu.VMEM((FINB, D), jnp.float32),
            pltpu.VMEM((FINB, 16), jnp.float32),
            pltpu.SemaphoreType.DMA,
            pltpu.SemaphoreType.DMA,
            pltpu.SemaphoreType.DMA,
            pltpu.SemaphoreType.DMA,
            pltpu.SemaphoreType.DMA,
            pltpu.SemaphoreType.DMA,
            pltpu.SemaphoreType.DMA,
        ],
    )
    def k(g_hbm, src_hbm, dst_hbm, scale_hbm, out_hbm,
          acc, ssrc0, ssrc1, sdst0, sdst1, dloc0, dloc1,
          rows0, rows1, rows2, facc, fg, scbuf,
          sg0, sg1, sg2, ss0, ss1, ss2, si):
        c = lax.axis_index("c")
        s = lax.axis_index("s")
        nodebase = c * HALF
        rb = (rows0, rows1, rows2)
        sg = (sg0, sg1, sg2)
        ss = (ss0, ss1, ss2)
        ssrcs = (ssrc0, ssrc1)
        sdsts = (sdst0, sdst1)
        dlocs = (dloc0, dloc1)

        def zrow(i, _):
            for kk in range(D // 16):
                facc[i, pl.ds(kk * 16, 16)] = jnp.zeros((16,), jnp.float32)
            return 0

        lax.fori_loop(0, FINB, zrow, 0)
        for z in range(FIN // FINB):
            pltpu.sync_copy(facc, acc.at[pl.ds(s * FIN + z * FINB, FINB)])
        plsc.subcore_barrier()

        ebase = s * (NSB * SB)

        def stage(sb_idx, par):
            pltpu.async_copy(
                src_hbm.at[pl.ds(ebase + sb_idx * SB, SB)], ssrcs[par], si)
            pltpu.async_copy(
                dst_hbm.at[pl.ds(ebase + sb_idx * SB, SB)], sdsts[par], si)

        def stage_wait(sb_idx, par):
            pltpu.make_async_copy(
                src_hbm.at[pl.ds(ebase + sb_idx * SB, SB)], ssrcs[par], si
            ).wait()
            pltpu.make_async_copy(
                dst_hbm.at[pl.ds(ebase + sb_idx * SB, SB)], sdsts[par], si
            ).wait()

        def run_superbatch(sb_idx, par):
            ssrc = ssrcs[par]
            dloc = dlocs[par]
            sdst = sdsts[par]
            for b in range(SB):
                for kk in range(B // 16):
                    dvec = sdst[b, pl.ds(kk * 16, 16)]
                    inr = (dvec >= nodebase) & (dvec < nodebase + HALF)
                    dloc[b, pl.ds(kk * 16, 16)] = jnp.where(
                        inr, dvec - nodebase, TRASH)

            def gissue(b):
                return pltpu.async_copy(
                    g_hbm.at[ssrc.at[b]], rb[b % 3], sg[b % 3])

            def sissue(b):
                return pltpu.async_copy(
                    rb[b % 3], acc.at[dloc.at[b]], ss[b % 3], add=True)

            gd = {0: gissue(0), 1: gissue(1)}
            sd = {}
            for b in range(SB):
                gd[b].wait()
                sd[b] = sissue(b)
                if b + 2 < SB:
                    if b - 1 >= 0:
                        sd[b - 1].wait()
                    gd[b + 2] = gissue(b + 2)
            sd[SB - 3].wait()
            sd[SB - 2].wait()
            sd[SB - 1].wait()

        stage(0, 0)
        nhalf = NSB // 2

        def pair(m, _):
            stage_wait(2 * m, 0)

            @pl.when(m < nhalf)
            def _():
                stage(2 * m + 1, 1)

            run_superbatch(2 * m, 0)
            stage_wait(2 * m + 1, 1)

            @pl.when(m < nhalf - 1)
            def _():
                stage(2 * m + 2, 0)

            run_superbatch(2 * m + 1, 1)
            return 0

        lax.fori_loop(0, nhalf, pair, 0)
        plsc.subcore_barrier()

        def fin(off, sz):
            gbase = nodebase + s * FIN + off
            pltpu.sync_copy(acc.at[pl.ds(s * FIN + off, sz)], facc.at[pl.ds(0, sz)])
            pltpu.sync_copy(g_hbm.at[pl.ds(gbase, sz)], fg.at[pl.ds(0, sz)])
            pltpu.sync_copy(scale_hbm.at[pl.ds(gbase, sz)], scbuf.at[pl.ds(0, sz)])

            def frow(i, _):
                sc = scbuf[i, :][0]
                for kk in range(D // 16):
                    sl = pl.ds(kk * 16, 16)
                    facc[i, sl] = sc * (facc[i, sl] + fg[i, sl])
                return 0

            lax.fori_loop(0, sz, frow, 0)
            pltpu.sync_copy(facc.at[pl.ds(0, sz)], out_hbm.at[pl.ds(gbase, sz)])

        @pl.when(s == 15)
        def _():
            for z in range(3):
                fin(z * FINB, FINB)
            fin(3 * FINB, 8)

        @pl.when(s < 15)
        def _():
            for z in range(FIN // FINB):
                fin(z * FINB, FINB)

    return k


def _tc_in_call(x, W1, b1, dinv_col):
    D = x.shape[1]
    H = W1.shape[1]
    BM = 400

    def body(x_ref, w_ref, b_ref, dv_ref, o_ref):
        h = jnp.dot(x_ref[...], w_ref[...], preferred_element_type=jnp.float32)
        o_ref[...] = dv_ref[...] * (h + b_ref[...])

    return pl.pallas_call(
        body,
        out_shape=jax.ShapeDtypeStruct((N, H), jnp.float32),
        grid=(N // BM,),
        in_specs=[
            pl.BlockSpec((BM, D), lambda i: (i, 0)),
            pl.BlockSpec((D, H), lambda i: (0, 0)),
            pl.BlockSpec((1, H), lambda i: (0, 0)),
            pl.BlockSpec((BM, 1), lambda i: (i, 0)),
        ],
        out_specs=pl.BlockSpec((BM, H), lambda i: (i, 0)),
    )(x, W1, b1, dinv_col)


def _tc_out_call(g30, Wc, bc, dinv_col):
    H = g30.shape[1]
    C = Wc.shape[1]
    BM = 400

    def body(g_ref, w_ref, b_ref, dv_ref, o_ref):
        h = jax.nn.relu(g_ref[...] / dv_ref[...])
        o = jnp.dot(h, w_ref[...], preferred_element_type=jnp.float32)
        o_ref[...] = dv_ref[...] * (o + b_ref[...])

    return pl.pallas_call(
        body,
        out_shape=jax.ShapeDtypeStruct((N, C), jnp.float32),
        grid=(N // BM,),
        in_specs=[
            pl.BlockSpec((BM, H), lambda i: (i, 0)),
            pl.BlockSpec((H, C), lambda i: (0, 0)),
            pl.BlockSpec((1, C), lambda i: (0, 0)),
            pl.BlockSpec((BM, 1), lambda i: (i, 0)),
        ],
        out_specs=pl.BlockSpec((BM, C), lambda i: (i, 0)),
    )(g30, Wc, bc, dinv_col)


def kernel(x, edge_index, prop_nums, W1, b1, Wc, bc):
    D = x.shape[1]
    C = Wc.shape[1]
    E = edge_index.shape[1]
    src = edge_index[0].astype(jnp.int32)
    dst = edge_index[1].astype(jnp.int32)
    pad = E_PAD - E
    srcp = jnp.concatenate([src, jnp.zeros((pad,), jnp.int32)])
    dstp = jnp.concatenate([dst, jnp.full((pad,), N, jnp.int32)])
    src2d = srcp.reshape(EROWS, B)
    dst2d = dstp.reshape(EROWS, B)

    prop128 = _make_prop(D)
    b1r = b1.reshape(1, -1)
    bcr = bc.reshape(1, -1)

    def phase_deg(g, dv, d2):
        return (jnp.ones((N, D), jnp.float32),
                jnp.ones((N, 16), jnp.float32), dv, d2)

    def phase_init(g, dv, d2):
        dv2, d22 = _dinv_tc_call(g)
        gin = _tc_in_call(x, W1, b1r, dv2[:, :1])
        return gin, d22, dv2, d22

    def phase_mid(g, dv, d2):
        return g, d2, dv, d2

    def phase_last(g, dv, d2):
        q = _tc_out_call(g, Wc, bcr, dv[:, :1])
        return jnp.pad(q, ((0, 0), (0, D - C))), dv, dv, d2

    def loop_body(i, carry):
        g, dv, d2 = carry
        sel = jnp.where(i == 0, 0,
                        jnp.where(i == 1, 1,
                                  jnp.where(i == prop_nums + 1, 3, 2)))
        gin, scale, dv, d2 = lax.switch(
            sel, [phase_deg, phase_init, phase_mid, phase_last], g, dv, d2)
        return prop128(gin, src2d, dst2d, scale), dv, d2

    zeros16 = jnp.zeros((N, 16), jnp.float32)
    out_full, _, _ = lax.fori_loop(
        0, prop_nums + 2, loop_body, (x, zeros16, zeros16))
    return out_full[:, :C]


# speedup vs baseline: 13.5959x; 4.1807x over previous
import functools

import jax
import jax.numpy as jnp
from jax import lax
from jax.experimental import pallas as pl
from jax.experimental.pallas import tpu as pltpu
from jax.experimental.pallas import tpu_sc as plsc

N = 10000
NPAD = 10240
DH = 64
SPR = NPAD // 16
FB = 128

B = 128
SB = 16
NSB = 10
EPT = B * SB * NSB
E_PAD = 16 * EPT
EROWS = E_PAD // B

_mesh = lambda: plsc.VectorSubcoreMesh(core_axis_name="c", subcore_axis_name="s")


def _make_prop():
    @functools.partial(
        pl.kernel,
        out_type=jax.ShapeDtypeStruct((2, NPAD, DH), jnp.float32),
        mesh=_mesh(),
        scratch_types=[
            pltpu.VMEM_SHARED((NPAD, DH), jnp.float32),
            pltpu.VMEM_SHARED((NPAD, DH), jnp.float32),
            pltpu.VMEM((SB, B), jnp.int32),
            pltpu.VMEM((SB, B), jnp.int32),
            pltpu.VMEM((SB, B), jnp.int32),
            pltpu.VMEM((SB, B), jnp.int32),
            pltpu.VMEM((B, DH), jnp.float32),
            pltpu.VMEM((B, DH), jnp.float32),
            pltpu.SemaphoreType.DMA,
            pltpu.SemaphoreType.DMA,
            pltpu.SemaphoreType.DMA,
            pltpu.SemaphoreType.DMA,
            pltpu.SemaphoreType.DMA,
        ],
    )
    def k(gpair_hbm, src_hbm, dst_hbm, scale_hbm, out_hbm,
          gsp, acc, ssrc0, ssrc1, sdst0, sdst1,
          rows0, rows1,
          sg0, sg1, ss0, ss1, si):
        c = lax.axis_index("c")
        s = lax.axis_index("s")
        rb = (rows0, rows1)
        sg = (sg0, sg1)
        ss = (ss0, ss1)
        ssrcs = (ssrc0, ssrc1)
        sdsts = (sdst0, sdst1)
        ghalf = gpair_hbm.at[c]
        rbase = s * SPR

        def zrow(i, _):
            for kk in range(DH // 16):
                rows1[i, pl.ds(kk * 16, 16)] = jnp.zeros((16,), jnp.float32)
            return 0

        lax.fori_loop(0, FB, zrow, 0)
        for z in range(SPR // FB):
            pltpu.sync_copy(rows1, acc.at[pl.ds(rbase + z * FB, FB)])
        for z in range(SPR // FB):
            sl = pl.ds(rbase + z * FB, FB)
            pltpu.sync_copy(ghalf.at[sl], rows0)
            pltpu.sync_copy(rows0, gsp.at[sl])
        plsc.subcore_barrier()

        ebase = s * (NSB * SB)

        def stage(sb_idx, par):
            pltpu.async_copy(
                src_hbm.at[pl.ds(ebase + sb_idx * SB, SB)], ssrcs[par], si)
            pltpu.async_copy(
                dst_hbm.at[pl.ds(ebase + sb_idx * SB, SB)], sdsts[par], si)

        def stage_wait(sb_idx, par):
            pltpu.make_async_copy(
                src_hbm.at[pl.ds(ebase + sb_idx * SB, SB)], ssrcs[par], si
            ).wait()
            pltpu.make_async_copy(
                dst_hbm.at[pl.ds(ebase + sb_idx * SB, SB)], sdsts[par], si
            ).wait()

        def run_superbatch(par):
            ssrc = ssrcs[par]
            sdst = sdsts[par]

            def gissue(b):
                return pltpu.async_copy(
                    gsp.at[ssrc.at[b]], rb[b % 2], sg[b % 2])

            def sissue(b):
                return pltpu.async_copy(
                    rb[b % 2], acc.at[sdst.at[b]], ss[b % 2], add=True)

            gd = {0: gissue(0)}
            sd = {}
            for b in range(SB):
                gd[b].wait()
                sd[b] = sissue(b)
                if b + 1 < SB:
                    if b - 1 >= 0:
                        sd[b - 1].wait()
                    gd[b + 1] = gissue(b + 1)
            sd[SB - 2].wait()
            sd[SB - 1].wait()

        stage(0, 0)
        nhalf = NSB // 2

        def pair(m, _):
            stage_wait(2 * m, 0)
            stage(2 * m + 1, 1)
            run_superbatch(0)
            stage_wait(2 * m + 1, 1)

            @pl.when(m < nhalf - 1)
            def _():
                stage(2 * m + 2, 0)

            run_superbatch(1)
            return 0

        lax.fori_loop(0, nhalf, pair, 0)
        plsc.subcore_barrier()

        def fin(z):
            sl = pl.ds(rbase + z * FB, FB)
            pltpu.sync_copy(acc.at[sl], rows0)
            pltpu.sync_copy(ghalf.at[sl], rows1)

            def fadd(i, _):
                for kk in range(DH // 16):
                    cs = pl.ds(kk * 16, 16)
                    rows0[i, cs] = rows0[i, cs] + rows1[i, cs]
                return 0

            lax.fori_loop(0, FB, fadd, 0)
            pltpu.sync_copy(scale_hbm.at[sl], rows1)

            def fmul(i, _):
                sc = rows1[i, :][0]
                for kk in range(DH // 16):
                    cs = pl.ds(kk * 16, 16)
                    rows0[i, cs] = sc * rows0[i, cs]
                return 0

            lax.fori_loop(0, FB, fmul, 0)
            pltpu.sync_copy(rows0, out_hbm.at[c].at[sl])

        for z in range(SPR // FB):
            fin(z)

    return k


def _dinv_tc_call(degpair):
    BM = 1280

    def body(deg_ref, dinv_ref, d2_ref):
        y = lax.rsqrt(deg_ref[0])
        dinv_ref[...] = y
        d2_ref[...] = y * y

    return pl.pallas_call(
        body,
        out_shape=(
            jax.ShapeDtypeStruct((NPAD, DH), jnp.float32),
            jax.ShapeDtypeStruct((NPAD, DH), jnp.float32),
        ),
        grid=(NPAD // BM,),
        in_specs=[pl.BlockSpec((1, BM, DH), lambda i: (0, i, 0))],
        out_specs=(
            pl.BlockSpec((BM, DH), lambda i: (i, 0)),
            pl.BlockSpec((BM, DH), lambda i: (i, 0)),
        ),
    )(degpair)


def _tc_in_call(xpad, W1pair, b1pair, dinv_col):
    D = xpad.shape[1]
    BM = 320

    def body(x_ref, w_ref, b_ref, dv_ref, o_ref):
        h = jnp.dot(x_ref[...], w_ref[0],
                    preferred_element_type=jnp.float32)
        o_ref[0] = dv_ref[...] * (h + b_ref[0])

    return pl.pallas_call(
        body,
        out_shape=jax.ShapeDtypeStruct((2, NPAD, DH), jnp.float32),
        grid=(2, NPAD // BM),
        in_specs=[
            pl.BlockSpec((BM, D), lambda j, i: (i, 0)),
            pl.BlockSpec((1, D, DH), lambda j, i: (j, 0, 0)),
            pl.BlockSpec((1, 1, DH), lambda j, i: (j, 0, 0)),
            pl.BlockSpec((BM, 1), lambda j, i: (i, 0)),
        ],
        out_specs=pl.BlockSpec((1, BM, DH), lambda j, i: (j, i, 0)),
    )(xpad, W1pair, b1pair, dinv_col)


def _tc_out_call(gpair, Wc, bc, dinv_col):
    C = Wc.shape[1]
    BM = 320

    def body(g_ref, w_ref, b_ref, dv_ref, o_ref):
        g2 = g_ref[...]
        g = jnp.concatenate([g2[0], g2[1]], axis=1)
        dv = dv_ref[...]
        h = jax.nn.relu(g / dv)
        q = dv * (jnp.dot(h, w_ref[...], preferred_element_type=jnp.float32)
                  + b_ref[...])
        o_ref[0] = jnp.concatenate(
            [q, jnp.zeros((BM, DH - C), jnp.float32)], axis=1)
        o_ref[1] = jnp.zeros((BM, DH), jnp.float32)

    return pl.pallas_call(
        body,
        out_shape=jax.ShapeDtypeStruct((2, NPAD, DH), jnp.float32),
        grid=(NPAD // BM,),
        in_specs=[
            pl.BlockSpec((2, BM, DH), lambda i: (0, i, 0)),
            pl.BlockSpec((2 * DH, C), lambda i: (0, 0)),
            pl.BlockSpec((1, C), lambda i: (0, 0)),
            pl.BlockSpec((BM, 1), lambda i: (i, 0)),
        ],
        out_specs=pl.BlockSpec((2, BM, DH), lambda i: (0, i, 0)),
    )(gpair, Wc, bc, dinv_col)


def kernel(x, edge_index, prop_nums, W1, b1, Wc, bc):
    C = Wc.shape[1]
    E = edge_index.shape[1]
    src = edge_index[0].astype(jnp.int32)
    dst = edge_index[1].astype(jnp.int32)
    pad = E_PAD - E
    srcp = jnp.concatenate([src, jnp.zeros((pad,), jnp.int32)])
    dstp = jnp.concatenate([dst, jnp.full((pad,), N, jnp.int32)])
    src2d = srcp.reshape(EROWS, B)
    dst2d = dstp.reshape(EROWS, B)
    xpad = jnp.pad(x, ((0, NPAD - N), (0, 0)))
    W1pair = W1.reshape(-1, 2, DH).transpose(1, 0, 2)
    b1pair = b1.reshape(2, 1, DH)
    bcr = bc.reshape(1, -1)

    prop = _make_prop()

    def phase_deg(g, dv, d2):
        return (jnp.ones((2, NPAD, DH), jnp.float32),
                jnp.ones((NPAD, DH), jnp.float32), dv, d2)

    def phase_init(g, dv, d2):
        dv2, d22 = _dinv_tc_call(g)
        gin = _tc_in_call(xpad, W1pair, b1pair, dv2[:, :1])
        return gin, d22, dv2, d22

    def phase_mid(g, dv, d2):
        return g, d2, dv, d2

    def phase_last(g, dv, d2):
        qpair = _tc_out_call(g, Wc, bcr, dv[:, :1])
        return qpair, dv, dv, d2

    def loop_body(i, carry):
        g, dv, d2 = carry
        sel = jnp.where(i == 0, 0,
                        jnp.where(i == 1, 1,
                                  jnp.where(i == prop_nums + 1, 3, 2)))
        gin, scale, dv, d2 = lax.switch(
            sel, [phase_deg, phase_init, phase_mid, phase_last], g, dv, d2)
        return prop(gin, src2d, dst2d, scale), dv, d2

    zscale = jnp.zeros((NPAD, DH), jnp.float32)
    out_full, _, _ = lax.fori_loop(
        0, prop_nums + 2, loop_body,
        (jnp.zeros((2, NPAD, DH), jnp.float32), zscale, zscale))
    return out_full[0][:N, :C]
